# Initial kernel scaffold; baseline (speedup 1.0000x reference)
import jax
import jax.numpy as jnp
from jax.experimental import pallas as pl


def kernel(x, cu_seqlens, W_base, b_base, W_p, b_p, W_r, b_r, w_att_p, W_q_p, w_att_r, W_q_r):
    raise NotImplementedError("write your pallas kernel here")


# speedup vs baseline: 15.9332x; 15.9332x over previous
import jax
import jax.numpy as jnp
from jax.experimental import pallas as pl

B = 16
N = 4096
D = 256


def _graph_kernel(x_ref, starts_ref, ends_ref, wb_ref, bb_ref, wp_ref, bp_ref,
                  wr_ref, br_ref, ap_ref, wqp_ref, ar_ref, wqr_ref,
                  keys_ref, pq_ref, rq_ref):
    x = x_ref[...]
    feats = jnp.maximum(jnp.dot(x, wb_ref[...],
                                preferred_element_type=jnp.float32)
                        + bb_ref[...], 0.0)

    ids = jax.lax.broadcasted_iota(jnp.int32, (B, N), 1)
    seg = jnp.logical_and(ids >= starts_ref[...], ids < ends_ref[...])
    maskf = seg.astype(jnp.float32)

    seg_sum = jnp.dot(maskf, feats, preferred_element_type=jnp.float32)
    denom = jnp.maximum(jnp.sum(maskf, axis=1, keepdims=True), 1.0)
    keys_ref[...] = seg_sum / denom

    def branch(w_ref, b_ref, att_ref, wq_ref, out_ref):
        feat = jnp.maximum(jnp.dot(feats, w_ref[...],
                                   preferred_element_type=jnp.float32)
                           + b_ref[...], 0.0)
        scores = jnp.dot(feat, att_ref[...],
                         preferred_element_type=jnp.float32)
        s2 = jnp.where(seg, scores.reshape(1, N), -jnp.inf)
        m = jnp.max(s2, axis=1, keepdims=True)
        e = jnp.where(seg, jnp.exp(s2 - m), 0.0)
        l = jnp.sum(e, axis=1, keepdims=True)
        attn = e / jnp.maximum(l, 1e-30)
        pooled = jnp.dot(attn, feat, preferred_element_type=jnp.float32)
        out_ref[...] = jnp.dot(pooled, wq_ref[...],
                               preferred_element_type=jnp.float32)

    branch(wp_ref, bp_ref, ap_ref, wqp_ref, pq_ref)
    branch(wr_ref, br_ref, ar_ref, wqr_ref, rq_ref)


def kernel(x, cu_seqlens, W_base, b_base, W_p, b_p, W_r, b_r,
           w_att_p, W_q_p, w_att_r, W_q_r):
    cu = cu_seqlens.astype(jnp.int32)
    starts = cu[:-1].reshape(B, 1)
    ends = cu[1:].reshape(B, 1)
    out_shape = (
        jax.ShapeDtypeStruct((B, D), jnp.float32),
        jax.ShapeDtypeStruct((B, D), jnp.float32),
        jax.ShapeDtypeStruct((B, D), jnp.float32),
    )
    return pl.pallas_call(
        _graph_kernel,
        out_shape=out_shape,
    )(x, starts, ends,
      W_base, b_base.reshape(1, D),
      W_p, b_p.reshape(1, D),
      W_r, b_r.reshape(1, D),
      w_att_p.reshape(D, 1), W_q_p,
      w_att_r.reshape(D, 1), W_q_r)


# speedup vs baseline: 20.0673x; 1.2595x over previous
import jax
import jax.numpy as jnp
from jax.experimental import pallas as pl

B = 16
N = 4096
D = 256


def _graph_kernel(x_ref, starts_ref, ends_ref, wb_ref, bb_ref, wp_ref, bp_ref,
                  wr_ref, br_ref, ap_ref, wqp_ref, ar_ref, wqr_ref,
                  keys_ref, pq_ref, rq_ref):
    x = x_ref[...]
    feats = jnp.maximum(jnp.dot(x, wb_ref[...],
                                preferred_element_type=jnp.float32)
                        + bb_ref[...], 0.0)

    ids = jax.lax.broadcasted_iota(jnp.int32, (B, N), 1)
    starts = starts_ref[...]
    ends = ends_ref[...]
    seg = jnp.logical_and(ids >= starts, ids < ends)
    maskf = seg.astype(jnp.float32)

    seg_sum = jnp.dot(maskf, feats, preferred_element_type=jnp.float32)
    inv_len = 1.0 / jnp.maximum((ends - starts).astype(jnp.float32), 1.0)
    keys_ref[...] = seg_sum * inv_len

    def branch(w_ref, b_ref, att_ref, wq_ref, out_ref):
        feat = jnp.maximum(jnp.dot(feats, w_ref[...],
                                   preferred_element_type=jnp.float32)
                           + b_ref[...], 0.0)
        scores = jax.lax.dot_general(
            att_ref[...], feat, (((1,), (1,)), ((), ())),
            preferred_element_type=jnp.float32)
        s2 = jnp.where(seg, scores, -jnp.inf)
        m = jnp.max(s2, axis=1, keepdims=True)
        e = jnp.exp(s2 - m)
        l = jnp.sum(e, axis=1, keepdims=True)
        attn = e * (1.0 / jnp.maximum(l, 1e-30))
        pooled = jnp.dot(attn, feat, preferred_element_type=jnp.float32)
        out_ref[...] = jnp.dot(pooled, wq_ref[...],
                               preferred_element_type=jnp.float32)

    branch(wp_ref, bp_ref, ap_ref, wqp_ref, pq_ref)
    branch(wr_ref, br_ref, ar_ref, wqr_ref, rq_ref)


def kernel(x, cu_seqlens, W_base, b_base, W_p, b_p, W_r, b_r,
           w_att_p, W_q_p, w_att_r, W_q_r):
    cu = cu_seqlens.astype(jnp.int32)
    starts = cu[:-1].reshape(B, 1)
    ends = cu[1:].reshape(B, 1)
    out_shape = (
        jax.ShapeDtypeStruct((B, D), jnp.float32),
        jax.ShapeDtypeStruct((B, D), jnp.float32),
        jax.ShapeDtypeStruct((B, D), jnp.float32),
    )
    return pl.pallas_call(
        _graph_kernel,
        out_shape=out_shape,
    )(x, starts, ends,
      W_base, b_base.reshape(1, D),
      W_p, b_p.reshape(1, D),
      W_r, b_r.reshape(1, D),
      w_att_p.reshape(1, D), W_q_p,
      w_att_r.reshape(1, D), W_q_r)
